# baseline (speedup 1.0000x reference)
import functools

import jax
import jax.numpy as jnp
from jax import lax
from jax.experimental import pallas as pl
from jax.experimental.pallas import tpu as pltpu
from jax.experimental.pallas import tpu_sc as plsc

VOCAB = 1000000
D = 64
B_TOTAL = 4096 * 200

NC = 2
NS = 16
NW = NC * NS

CHUNK = 128
PER_W = B_TOTAL // NW
N_CHUNKS = PER_W // CHUNK


def _make_kernel():
  mesh = plsc.VectorSubcoreMesh(core_axis_name="c", subcore_axis_name="s")

  @functools.partial(
      pl.kernel,
      mesh=mesh,
      compiler_params=pltpu.CompilerParams(use_tc_tiling_on_sc=False),
      out_type=jax.ShapeDtypeStruct((B_TOTAL, D), jnp.float32),
      scratch_types=[
          pltpu.VMEM((N_CHUNKS, CHUNK), jnp.int32),
          pltpu.VMEM((CHUNK, D), jnp.float32),
          pltpu.SemaphoreType.DMA,
      ],
  )
  def emb_kernel(idx_hbm, table_hbm, out_hbm, idx_v, rows_v, sem):
    wid = lax.axis_index("s") * NC + lax.axis_index("c")
    base = wid * PER_W

    pltpu.sync_copy(idx_hbm.at[wid], idx_v)

    def body(j, _):
      pltpu.async_copy(table_hbm.at[idx_v.at[j]], rows_v, sem).wait()
      pltpu.sync_copy(rows_v, out_hbm.at[pl.ds(base + j * CHUNK, CHUNK)])
      return 0

    lax.fori_loop(0, N_CHUNKS, body, 0)

  return emb_kernel


_emb = _make_kernel()


@jax.jit
def kernel(x, weight):
  idx = x.astype(jnp.int32).reshape(NW, N_CHUNKS, CHUNK)
  out = _emb(idx, weight)
  return out.reshape(x.shape[0], x.shape[1], D)


# speedup vs baseline: 1.1172x; 1.1172x over previous
import functools

import jax
import jax.numpy as jnp
from jax import lax
from jax.experimental import pallas as pl
from jax.experimental.pallas import tpu as pltpu
from jax.experimental.pallas import tpu_sc as plsc

VOCAB = 1000000
D = 64
B_TOTAL = 4096 * 200

NC = 2
NS = 16
NW = NC * NS

CHUNK = 128
PER_W = B_TOTAL // NW
N_CHUNKS = PER_W // CHUNK


NBUF = 8
N_ITER = N_CHUNKS // NBUF


def _make_kernel():
  mesh = plsc.VectorSubcoreMesh(core_axis_name="c", subcore_axis_name="s")

  @functools.partial(
      pl.kernel,
      mesh=mesh,
      compiler_params=pltpu.CompilerParams(use_tc_tiling_on_sc=False),
      out_type=jax.ShapeDtypeStruct((B_TOTAL, D), jnp.float32),
      scratch_types=[
          pltpu.VMEM((N_CHUNKS, CHUNK), jnp.int32),
          pltpu.VMEM((NBUF, CHUNK, D), jnp.float32),
          pltpu.SemaphoreType.DMA((NBUF,)),
          pltpu.SemaphoreType.DMA((NBUF,)),
      ],
  )
  def emb_kernel(idx_hbm, table_hbm, out_hbm, idx_v, rows_v, gsem, wsem):
    wid = lax.axis_index("s") * NC + lax.axis_index("c")
    base = wid * PER_W

    pltpu.sync_copy(idx_hbm.at[wid], idx_v)

    def gather(j, b):
      return pltpu.make_async_copy(
          table_hbm.at[idx_v.at[j]], rows_v.at[b], gsem.at[b])

    def write(j, b):
      return pltpu.make_async_copy(
          rows_v.at[b], out_hbm.at[pl.ds(base + j * CHUNK, CHUNK)],
          wsem.at[b])

    for b in range(NBUF):
      gather(b, b).start()

    def body(i, _):
      for b in range(NBUF):
        j = i * NBUF + b
        gather(j, b).wait()
        write(j, b).start()
        jn = j + NBUF

        @pl.when(i < N_ITER - 1)
        def _():
          write(j, b).wait()
          gather(jn, b).start()

      return 0

    lax.fori_loop(0, N_ITER, body, 0)

    for b in range(NBUF):
      write(N_CHUNKS - NBUF + b, b).wait()

  return emb_kernel


_emb = _make_kernel()


@jax.jit
def kernel(x, weight):
  idx = x.astype(jnp.int32).reshape(NW, N_CHUNKS, CHUNK)
  out = _emb(idx, weight)
  return out.reshape(x.shape[0], x.shape[1], D)
